# baseline (speedup 1.0000x reference)
import functools

import jax
import jax.numpy as jnp
from jax import lax
from jax.experimental import pallas as pl
from jax.experimental.pallas import tpu as pltpu
from jax.experimental.pallas import tpu_sc as plsc

N = 10000
E = 320000
D = 128
EPS = 1e-9

NC = 2
NS = 16
NW = NC * NS
CHUNK = 32
EPAD = 327680
EPT = EPAD // NW
CPT = EPT // CHUNK
NBUF = 6
LOOK = 4
BLK = 16
NBLK = CPT // BLK
CPT_C0 = 144
RPT = 640
NPAD = NS * RPT

_mesh = plsc.VectorSubcoreMesh(core_axis_name="c", subcore_axis_name="s",
                               num_cores=NC, num_subcores=NS)


@functools.partial(
    pl.kernel,
    out_type=jax.ShapeDtypeStruct((NC * NPAD,), jnp.float32),
    mesh=_mesh,
    scratch_types=[
        pltpu.VMEM((CPT, CHUNK), jnp.int32),
        pltpu.VMEM((CPT, CHUNK), jnp.float32),
        pltpu.VMEM((640,), jnp.float32),
        pltpu.VMEM_SHARED((NPAD,), jnp.float32),
        pltpu.SemaphoreType.DMA,
    ],
)
def _deg_kernel(row2_hbm, val2_hbm, out_hbm, idx2d, val2d, zbuf, deg_sh, sem):
    c = lax.axis_index("c")
    s = lax.axis_index("s")
    wid = s * NC + c

    zeros16 = jnp.zeros((16,), jnp.float32)

    def zero_body(i, carry):
        zbuf[pl.ds(i * 16, 16)] = zeros16
        return carry

    lax.fori_loop(0, 40, zero_body, 0)
    pltpu.sync_copy(zbuf, deg_sh.at[pl.ds(s * RPT, RPT)])
    pltpu.sync_copy(row2_hbm.at[pl.ds(wid * CPT, CPT)], idx2d)
    pltpu.sync_copy(val2_hbm.at[pl.ds(wid * CPT, CPT)], val2d)
    plsc.subcore_barrier()

    FIRE = 32

    def outer(g, carry):
        def fire(i, c2):
            k = g * FIRE + i
            pltpu.async_copy(val2d.at[k], deg_sh.at[idx2d.at[k]], sem,
                             add=True)
            return c2

        lax.fori_loop(0, FIRE, fire, 0)

        def drain(i, c2):
            pltpu.make_async_copy(val2d.at[0], deg_sh.at[pl.ds(0, CHUNK)],
                                  sem).wait()
            return c2

        lax.fori_loop(0, FIRE, drain, 0)
        return carry

    lax.fori_loop(0, CPT // FIRE, outer, 0)
    plsc.subcore_barrier()
    pltpu.sync_copy(deg_sh.at[pl.ds(s * RPT, RPT)],
                    out_hbm.at[pl.ds(c * NPAD + s * RPT, RPT)])


@functools.partial(
    pl.kernel,
    out_type=jax.ShapeDtypeStruct((NC, NPAD, D), jnp.float32),
    mesh=_mesh,
    scratch_types=[
        pltpu.VMEM((2, BLK, CHUNK), jnp.int32),
        pltpu.VMEM((2, BLK, CHUNK), jnp.int32),
        pltpu.VMEM((2 * BLK, CHUNK), jnp.float32),
        pltpu.VMEM((NBUF * CHUNK, D), jnp.float32),
        pltpu.VMEM((8, D), jnp.float32),
        pltpu.VMEM_SHARED((NPAD, D), jnp.float32),
        pltpu.SemaphoreType.DMA((NBUF,)),
        pltpu.SemaphoreType.DMA((NBUF,)),
        pltpu.SemaphoreType.DMA,
        pltpu.SemaphoreType.DMA,
    ],
)
def _scatter_kernel(row2_hbm, col2_hbm, val2_hbm, y_hbm, out_hbm,
                    rstg, cstg, vstg, bufs, zrow, s_sh,
                    gsem, ssem, stsem, msem):
    c = lax.axis_index("c")
    s = lax.axis_index("s")
    cnt = jnp.where(c == 0, CPT_C0, 2 * CPT - CPT_C0)
    gbase = s * (2 * CPT) + c * CPT_C0

    zeros16 = jnp.zeros((16,), jnp.float32)

    def zero_body(r, carry):
        for f in range(D // 16):
            zrow[r, pl.ds(f * 16, 16)] = zeros16
        return carry

    lax.fori_loop(0, 8, zero_body, 0)

    def zero_fire(i, carry):
        pltpu.async_copy(zrow, s_sh.at[pl.ds(s * RPT + i * 8, 8)], msem)
        return carry

    lax.fori_loop(0, RPT // 8, zero_fire, 0)

    def zero_drain(i, carry):
        pltpu.make_async_copy(zrow, s_sh.at[pl.ds(0, 8)], msem).wait()
        return carry

    lax.fori_loop(0, RPT // 8, zero_drain, 0)

    pltpu.sync_copy(row2_hbm.at[pl.ds(gbase, BLK)], rstg.at[0])
    pltpu.sync_copy(col2_hbm.at[pl.ds(gbase, BLK)], cstg.at[0])
    pltpu.sync_copy(val2_hbm.at[pl.ds(gbase, BLK)], vstg.at[pl.ds(0, BLK)])
    plsc.subcore_barrier()
    for i in range(LOOK):
        pltpu.async_copy(y_hbm.at[cstg.at[0, i]],
                         bufs.at[pl.ds(i * CHUNK, CHUNK)], gsem.at[i])

    def slot(k, carry):
        b = lax.rem(k, NBUF)
        blk = lax.div(k, BLK)
        p = lax.rem(blk, 2)
        kk = lax.rem(k, BLK)

        @pl.when(k >= 2)
        def _drain_sc():
            pltpu.make_async_copy(bufs.at[pl.ds(0, CHUNK)],
                                  s_sh.at[pl.ds(0, CHUNK)],
                                  ssem.at[lax.rem(k - 2, NBUF)]).wait()

        @pl.when(jnp.logical_and(kk == 1, k <= cnt - BLK))
        def _prefetch():
            pn = 1 - p
            nb = (blk + 1) * BLK
            pltpu.async_copy(row2_hbm.at[pl.ds(gbase + nb, BLK)],
                             rstg.at[pn], stsem)
            pltpu.async_copy(col2_hbm.at[pl.ds(gbase + nb, BLK)],
                             cstg.at[pn], stsem)
            pltpu.async_copy(val2_hbm.at[pl.ds(gbase + nb, BLK)],
                             vstg.at[pl.ds(pn * BLK, BLK)], stsem)

        @pl.when(k <= cnt - 1 - LOOK)
        def _fire_g():
            kn = k + LOOK
            pn2 = lax.rem(lax.div(kn, BLK), 2)

            @pl.when(lax.rem(kn, BLK) == 0)
            def _wait_stg():
                for _ in range(3):
                    pltpu.make_async_copy(row2_hbm.at[pl.ds(0, BLK)],
                                          rstg.at[0], stsem).wait()

            bn = lax.rem(kn, NBUF)
            pltpu.async_copy(y_hbm.at[cstg.at[pn2, lax.rem(kn, BLK)]],
                             bufs.at[pl.ds(bn * CHUNK, CHUNK)],
                             gsem.at[bn])

        pltpu.make_async_copy(y_hbm.at[pl.ds(0, CHUNK)],
                              bufs.at[pl.ds(0, CHUNK)], gsem.at[b]).wait()

        vrow = p * BLK + kk
        brow = b * CHUNK
        for j in range(CHUNK // 16):
            v16 = vstg[vrow, pl.ds(j * 16, 16)]
            for l in range(16):
                a = jnp.full((16,), v16[l], jnp.float32)
                e = brow + j * 16 + l
                for f in range(D // 16):
                    bufs[e, pl.ds(f * 16, 16)] = bufs[e, pl.ds(f * 16, 16)] * a

        pltpu.async_copy(bufs.at[pl.ds(brow, CHUNK)],
                         s_sh.at[rstg.at[p, kk]], ssem.at[b], add=True)
        return carry

    lax.fori_loop(0, cnt, slot, 0)

    for dk in (2, 1):
        pltpu.make_async_copy(bufs.at[pl.ds(0, CHUNK)],
                              s_sh.at[pl.ds(0, CHUNK)],
                              ssem.at[lax.rem(cnt - dk, NBUF)]).wait()
    plsc.subcore_barrier()
    pltpu.sync_copy(s_sh.at[pl.ds(s * RPT, RPT)],
                    out_hbm.at[c, pl.ds(s * RPT, RPT)])


def _prep_body(degp_ref, x_ref, dis_ref, y_ref):
    deg = degp_ref[:, 0:1] + degp_ref[:, 1:2]
    dis = lax.rsqrt(jnp.maximum(deg, EPS))
    dis_ref[...] = dis
    y_ref[...] = x_ref[...] * dis


def _tc_prep(degp_t, x):
    blk = 1000
    return pl.pallas_call(
        _prep_body,
        grid=(N // blk,),
        in_specs=[
            pl.BlockSpec((blk, 2), lambda i: (i, 0)),
            pl.BlockSpec((blk, D), lambda i: (i, 0)),
        ],
        out_specs=[
            pl.BlockSpec((blk, 1), lambda i: (i, 0)),
            pl.BlockSpec((blk, D), lambda i: (i, 0)),
        ],
        out_shape=[
            jax.ShapeDtypeStruct((N, 1), jnp.float32),
            jax.ShapeDtypeStruct((N, D), jnp.float32),
        ],
    )(degp_t, x)


def _mm_body(s0_ref, s1_ref, dis_ref, w_ref, out_ref):
    ax = (s0_ref[...] + s1_ref[...]) * dis_ref[...]
    h = lax.dot_general(ax, w_ref[...], (((1,), (1,)), ((), ())),
                        preferred_element_type=jnp.float32)
    out_ref[...] = jnp.maximum(h, 0.0)


def _tc_matmul(s0, s1, dis, w):
    blk = 1000
    return pl.pallas_call(
        _mm_body,
        grid=(N // blk,),
        in_specs=[
            pl.BlockSpec((blk, D), lambda i: (i, 0)),
            pl.BlockSpec((blk, D), lambda i: (i, 0)),
            pl.BlockSpec((blk, 1), lambda i: (i, 0)),
            pl.BlockSpec((D, D), lambda i: (0, 0)),
        ],
        out_specs=pl.BlockSpec((blk, D), lambda i: (i, 0)),
        out_shape=jax.ShapeDtypeStruct((N, D), jnp.float32),
    )(s0, s1, dis, w)


def kernel(X, A_indices, A_values, W):
    pad = EPAD - E
    row = jnp.concatenate([A_indices[0], jnp.zeros((pad,), jnp.int32)])
    col = jnp.concatenate([A_indices[1], jnp.zeros((pad,), jnp.int32)])
    val = jnp.concatenate([A_values, jnp.zeros((pad,), jnp.float32)])
    row2 = row.reshape(EPAD // CHUNK, CHUNK)
    col2 = col.reshape(EPAD // CHUNK, CHUNK)
    val2 = val.reshape(EPAD // CHUNK, CHUNK)

    deg_p = _deg_kernel(row2, val2).reshape(NC, NPAD)
    degp_t = jnp.transpose(deg_p)[:N, :]
    dis, Y = _tc_prep(degp_t, X)
    s_p = _scatter_kernel(row2, col2, val2, Y)
    return _tc_matmul(s_p[0, :N, :], s_p[1, :N, :], dis, W)


# speedup vs baseline: 1.1588x; 1.1588x over previous
import functools

import jax
import jax.numpy as jnp
from jax import lax
from jax.experimental import pallas as pl
from jax.experimental.pallas import tpu as pltpu
from jax.experimental.pallas import tpu_sc as plsc

N = 10000
E = 320000
D = 128
EPS = 1e-9

NC = 2
NS = 16
NW = NC * NS
CHUNK = 32
EPAD = 327680
EPT = EPAD // NW
CPT = EPT // CHUNK
NBUF = 6
LOOK = 4
BLK = 16
NBLK = CPT // BLK
CPT_C0 = 496
RPT = 640
NPAD = NS * RPT

_mesh = plsc.VectorSubcoreMesh(core_axis_name="c", subcore_axis_name="s",
                               num_cores=NC, num_subcores=NS)


@functools.partial(
    pl.kernel,
    out_type=jax.ShapeDtypeStruct((NC * NPAD,), jnp.float32),
    mesh=_mesh,
    scratch_types=[
        pltpu.VMEM((CPT, CHUNK), jnp.int32),
        pltpu.VMEM((CPT, CHUNK), jnp.float32),
        pltpu.VMEM((640,), jnp.float32),
        pltpu.VMEM_SHARED((NPAD,), jnp.float32),
        pltpu.SemaphoreType.DMA,
    ],
)
def _deg_kernel(row2_hbm, val2_hbm, out_hbm, idx2d, val2d, zbuf, deg_sh, sem):
    c = lax.axis_index("c")
    s = lax.axis_index("s")
    wid = s * NC + c

    zeros16 = jnp.zeros((16,), jnp.float32)

    def zero_body(i, carry):
        zbuf[pl.ds(i * 16, 16)] = zeros16
        return carry

    lax.fori_loop(0, 40, zero_body, 0)
    pltpu.sync_copy(zbuf, deg_sh.at[pl.ds(s * RPT, RPT)])
    pltpu.sync_copy(row2_hbm.at[pl.ds(wid * CPT, CPT)], idx2d)
    pltpu.sync_copy(val2_hbm.at[pl.ds(wid * CPT, CPT)], val2d)
    plsc.subcore_barrier()

    FIRE = 32

    def outer(g, carry):
        def fire(i, c2):
            k = g * FIRE + i
            pltpu.async_copy(val2d.at[k], deg_sh.at[idx2d.at[k]], sem,
                             add=True)
            return c2

        lax.fori_loop(0, FIRE, fire, 0)

        def drain(i, c2):
            pltpu.make_async_copy(val2d.at[0], deg_sh.at[pl.ds(0, CHUNK)],
                                  sem).wait()
            return c2

        lax.fori_loop(0, FIRE, drain, 0)
        return carry

    lax.fori_loop(0, CPT // FIRE, outer, 0)
    plsc.subcore_barrier()
    pltpu.sync_copy(deg_sh.at[pl.ds(s * RPT, RPT)],
                    out_hbm.at[pl.ds(c * NPAD + s * RPT, RPT)])


@functools.partial(
    pl.kernel,
    out_type=jax.ShapeDtypeStruct((NC, NPAD, D), jnp.float32),
    mesh=_mesh,
    scratch_types=[
        pltpu.VMEM((2, BLK, CHUNK), jnp.int32),
        pltpu.VMEM((2, BLK, CHUNK), jnp.int32),
        pltpu.VMEM((2 * BLK, CHUNK), jnp.float32),
        pltpu.VMEM((NBUF * CHUNK, D), jnp.float32),
        pltpu.VMEM((8, D), jnp.float32),
        pltpu.VMEM_SHARED((NPAD, D), jnp.float32),
        pltpu.SemaphoreType.DMA((NBUF,)),
        pltpu.SemaphoreType.DMA((NBUF,)),
        pltpu.SemaphoreType.DMA,
        pltpu.SemaphoreType.DMA,
    ],
)
def _scatter_kernel(row2_hbm, col2_hbm, val2_hbm, y_hbm, out_hbm,
                    rstg, cstg, vstg, bufs, zrow, s_sh,
                    gsem, ssem, stsem, msem):
    c = lax.axis_index("c")
    s = lax.axis_index("s")
    cnt = jnp.where(c == 0, CPT_C0, 2 * CPT - CPT_C0)
    gbase = s * (2 * CPT) + c * CPT_C0

    zeros16 = jnp.zeros((16,), jnp.float32)

    def zero_body(r, carry):
        for f in range(D // 16):
            zrow[r, pl.ds(f * 16, 16)] = zeros16
        return carry

    lax.fori_loop(0, 8, zero_body, 0)

    def zero_fire(i, carry):
        pltpu.async_copy(zrow, s_sh.at[pl.ds(s * RPT + i * 8, 8)], msem)
        return carry

    lax.fori_loop(0, RPT // 8, zero_fire, 0)

    def zero_drain(i, carry):
        pltpu.make_async_copy(zrow, s_sh.at[pl.ds(0, 8)], msem).wait()
        return carry

    lax.fori_loop(0, RPT // 8, zero_drain, 0)

    pltpu.sync_copy(row2_hbm.at[pl.ds(gbase, BLK)], rstg.at[0])
    pltpu.sync_copy(col2_hbm.at[pl.ds(gbase, BLK)], cstg.at[0])
    pltpu.sync_copy(val2_hbm.at[pl.ds(gbase, BLK)], vstg.at[pl.ds(0, BLK)])
    plsc.subcore_barrier()
    for i in range(LOOK):
        pltpu.async_copy(y_hbm.at[cstg.at[0, i]],
                         bufs.at[pl.ds(i * CHUNK, CHUNK)], gsem.at[i])

    def slot(k, carry):
        b = lax.rem(k, NBUF)
        blk = lax.div(k, BLK)
        p = lax.rem(blk, 2)
        kk = lax.rem(k, BLK)

        @pl.when(k >= 2)
        def _drain_sc():
            pltpu.make_async_copy(bufs.at[pl.ds(0, CHUNK)],
                                  s_sh.at[pl.ds(0, CHUNK)],
                                  ssem.at[lax.rem(k - 2, NBUF)]).wait()

        @pl.when(jnp.logical_and(kk == 1, k <= cnt - BLK))
        def _prefetch():
            pn = 1 - p
            nb = (blk + 1) * BLK
            pltpu.async_copy(row2_hbm.at[pl.ds(gbase + nb, BLK)],
                             rstg.at[pn], stsem)
            pltpu.async_copy(col2_hbm.at[pl.ds(gbase + nb, BLK)],
                             cstg.at[pn], stsem)
            pltpu.async_copy(val2_hbm.at[pl.ds(gbase + nb, BLK)],
                             vstg.at[pl.ds(pn * BLK, BLK)], stsem)

        @pl.when(k <= cnt - 1 - LOOK)
        def _fire_g():
            kn = k + LOOK
            pn2 = lax.rem(lax.div(kn, BLK), 2)

            @pl.when(lax.rem(kn, BLK) == 0)
            def _wait_stg():
                for _ in range(3):
                    pltpu.make_async_copy(row2_hbm.at[pl.ds(0, BLK)],
                                          rstg.at[0], stsem).wait()

            bn = lax.rem(kn, NBUF)
            pltpu.async_copy(y_hbm.at[cstg.at[pn2, lax.rem(kn, BLK)]],
                             bufs.at[pl.ds(bn * CHUNK, CHUNK)],
                             gsem.at[bn])

        pltpu.make_async_copy(y_hbm.at[pl.ds(0, CHUNK)],
                              bufs.at[pl.ds(0, CHUNK)], gsem.at[b]).wait()

        vrow = p * BLK + kk
        brow = b * CHUNK
        for j in range(CHUNK // 16):
            v16 = vstg[vrow, pl.ds(j * 16, 16)]
            for l in range(16):
                a = jnp.full((16,), v16[l], jnp.float32)
                e = brow + j * 16 + l
                for f in range(D // 16):
                    bufs[e, pl.ds(f * 16, 16)] = bufs[e, pl.ds(f * 16, 16)] * a

        pltpu.async_copy(bufs.at[pl.ds(brow, CHUNK)],
                         s_sh.at[rstg.at[p, kk]], ssem.at[b], add=True)
        return carry

    lax.fori_loop(0, cnt, slot, 0)

    for dk in (2, 1):
        pltpu.make_async_copy(bufs.at[pl.ds(0, CHUNK)],
                              s_sh.at[pl.ds(0, CHUNK)],
                              ssem.at[lax.rem(cnt - dk, NBUF)]).wait()
    plsc.subcore_barrier()
    pltpu.sync_copy(s_sh.at[pl.ds(s * RPT, RPT)],
                    out_hbm.at[c, pl.ds(s * RPT, RPT)])


def _prep_body(degp_ref, x_ref, dis_ref, y_ref):
    deg = degp_ref[:, 0:1] + degp_ref[:, 1:2]
    dis = lax.rsqrt(jnp.maximum(deg, EPS))
    dis_ref[...] = dis
    y_ref[...] = x_ref[...] * dis


def _tc_prep(degp_t, x):
    blk = 1000
    return pl.pallas_call(
        _prep_body,
        grid=(N // blk,),
        in_specs=[
            pl.BlockSpec((blk, 2), lambda i: (i, 0)),
            pl.BlockSpec((blk, D), lambda i: (i, 0)),
        ],
        out_specs=[
            pl.BlockSpec((blk, 1), lambda i: (i, 0)),
            pl.BlockSpec((blk, D), lambda i: (i, 0)),
        ],
        out_shape=[
            jax.ShapeDtypeStruct((N, 1), jnp.float32),
            jax.ShapeDtypeStruct((N, D), jnp.float32),
        ],
    )(degp_t, x)


def _mm_body(s0_ref, s1_ref, dis_ref, w_ref, out_ref):
    ax = (s0_ref[...] + s1_ref[...]) * dis_ref[...]
    h = lax.dot_general(ax, w_ref[...], (((1,), (1,)), ((), ())),
                        preferred_element_type=jnp.float32)
    out_ref[...] = jnp.maximum(h, 0.0)


def _tc_matmul(s0, s1, dis, w):
    blk = 1000
    return pl.pallas_call(
        _mm_body,
        grid=(N // blk,),
        in_specs=[
            pl.BlockSpec((blk, D), lambda i: (i, 0)),
            pl.BlockSpec((blk, D), lambda i: (i, 0)),
            pl.BlockSpec((blk, 1), lambda i: (i, 0)),
            pl.BlockSpec((D, D), lambda i: (0, 0)),
        ],
        out_specs=pl.BlockSpec((blk, D), lambda i: (i, 0)),
        out_shape=jax.ShapeDtypeStruct((N, D), jnp.float32),
    )(s0, s1, dis, w)


def kernel(X, A_indices, A_values, W):
    pad = EPAD - E
    row = jnp.concatenate([A_indices[0], jnp.zeros((pad,), jnp.int32)])
    col = jnp.concatenate([A_indices[1], jnp.zeros((pad,), jnp.int32)])
    val = jnp.concatenate([A_values, jnp.zeros((pad,), jnp.float32)])
    row2 = row.reshape(EPAD // CHUNK, CHUNK)
    col2 = col.reshape(EPAD // CHUNK, CHUNK)
    val2 = val.reshape(EPAD // CHUNK, CHUNK)

    deg_p = _deg_kernel(row2, val2).reshape(NC, NPAD)
    degp_t = jnp.transpose(deg_p)[:N, :]
    dis, Y = _tc_prep(degp_t, X)
    s_p = _scatter_kernel(row2, col2, val2, Y)
    return _tc_matmul(s_p[0, :N, :], s_p[1, :N, :], dis, W)
